# baseline (speedup 1.0000x reference)
import numpy as np
import jax
import jax.numpy as jnp
from jax import lax
from jax.experimental import pallas as pl
from jax.experimental.pallas import tpu as pltpu
from jax.experimental.pallas import tpu_sc as plsc

_VOCAB = 100000
_D = 512
_SEQ = 8192
_LANES = 16

_NC = 2
_NS = 16
_NW = _NC * _NS
_BPW = _SEQ // _NW
_C = 32
_NCHUNK = _BPW // _C


def _positional(seq_len, d):
    pos = np.arange(seq_len, dtype=np.float64)[:, None]
    hid = np.arange(d, dtype=np.float64)[None, :]
    angles = pos / np.power(10000.0, 2.0 * (np.floor(hid / 2.0)) / d)
    pe = np.array(angles)
    pe[:, 0::2] = np.sin(angles[:, 0::2])
    pe[:, 1::2] = np.cos(angles[:, 1::2])
    return pe.astype(np.float32)


_PE = _positional(_SEQ, _D)


def _body(x_hbm, table_hbm, pe_hbm, out_hbm, idx_v, rows_v, pe_v,
          sem_g, sem_p, sem_w):
    cid = lax.axis_index("c")
    sid = lax.axis_index("s")
    wid = sid * _NC + cid
    base0 = wid * _BPW

    pltpu.sync_copy(x_hbm.at[pl.ds(base0, _BPW)], idx_v)

    def start_in(j):
        b = j % 2
        g = pltpu.async_copy(table_hbm.at[idx_v.at[pl.ds(j * _C, _C)]],
                             rows_v.at[b], sem_g.at[b])
        p = pltpu.async_copy(pe_hbm.at[pl.ds((base0 + j * _C) * _D, _C * _D)],
                             pe_v.at[b], sem_p.at[b])
        return g, p

    inflight = {0: start_in(0)}
    wb = {}
    for j in range(_NCHUNK):
        b = j % 2
        if j - 1 >= 0:
            wb[j - 1].wait()
        if j + 1 < _NCHUNK:
            inflight[j + 1] = start_in(j + 1)
        g, p = inflight.pop(j)
        g.wait()
        p.wait()

        def add_row(r):
            for k in range(_D // _LANES):
                sl = pl.ds(k * _LANES, _LANES)
                pe_sl = pl.ds(r * _D + k * _LANES, _LANES)
                rows_v[b, r, sl] = rows_v[b, r, sl] + pe_v[b, pe_sl]

        lax.fori_loop(0, _C, lambda r, _: (add_row(r), 0)[1], 0,
                      unroll=2)

        wb[j] = pltpu.async_copy(rows_v.at[b],
                                 out_hbm.at[pl.ds(base0 + j * _C, _C)],
                                 sem_w.at[b])
    wb[_NCHUNK - 1].wait()


_sc_call = pl.kernel(
    _body,
    out_type=jax.ShapeDtypeStruct((_SEQ, _D), jnp.float32),
    mesh=plsc.VectorSubcoreMesh(core_axis_name="c", subcore_axis_name="s"),
    scratch_types=[
        pltpu.VMEM((_BPW,), jnp.int32),
        pltpu.VMEM((2, _C, _D), jnp.float32),
        pltpu.VMEM((2, _C * _D), jnp.float32),
        pltpu.SemaphoreType.DMA((2,)),
        pltpu.SemaphoreType.DMA((2,)),
        pltpu.SemaphoreType.DMA((2,)),
    ],
)


@jax.jit
def kernel(x, table):
    xi = x.astype(jnp.int32)
    return _sc_call(xi, table, jnp.asarray(_PE.reshape(-1)))


# speedup vs baseline: 1.3784x; 1.3784x over previous
import numpy as np
import jax
import jax.numpy as jnp
from jax import lax
from jax.experimental import pallas as pl
from jax.experimental.pallas import tpu as pltpu
from jax.experimental.pallas import tpu_sc as plsc

_VOCAB = 100000
_D = 512
_SEQ = 8192
_LANES = 16

_NC = 2
_NS = 16
_NW = _NC * _NS
_BPW = _SEQ // _NW
_C = 32
_NCHUNK = _BPW // _C


def _positional(seq_len, d):
    pos = np.arange(seq_len, dtype=np.float64)[:, None]
    hid = np.arange(d, dtype=np.float64)[None, :]
    angles = pos / np.power(10000.0, 2.0 * (np.floor(hid / 2.0)) / d)
    pe = np.array(angles)
    pe[:, 0::2] = np.sin(angles[:, 0::2])
    pe[:, 1::2] = np.cos(angles[:, 1::2])
    return pe.astype(np.float32)


_PE = _positional(_SEQ, _D)


def _body(x_hbm, table_hbm, pe_hbm, out_hbm, idx_v, rows_v, pe_v,
          sem_g, sem_p, sem_w):
    cid = lax.axis_index("c")
    sid = lax.axis_index("s")
    wid = sid * _NC + cid
    base0 = wid * _BPW

    pltpu.sync_copy(x_hbm.at[pl.ds(base0, _BPW)], idx_v)

    def start_in(j):
        b = j % 2
        g = pltpu.async_copy(table_hbm.at[idx_v.at[pl.ds(j * _C, _C)]],
                             rows_v.at[b], sem_g.at[b])
        p = pltpu.async_copy(pe_hbm.at[pl.ds(base0 + j * _C, _C)],
                             pe_v.at[b], sem_p.at[b])
        return g, p

    inflight = {0: start_in(0)}
    wb = {}
    for j in range(_NCHUNK):
        b = j % 2
        if j - 1 >= 0:
            wb[j - 1].wait()
        if j + 1 < _NCHUNK:
            inflight[j + 1] = start_in(j + 1)
        g, p = inflight.pop(j)
        g.wait()
        p.wait()

        def add_row(r):
            for k in range(_D // _LANES):
                sl = pl.ds(k * _LANES, _LANES)
                rows_v[b, r, sl] = rows_v[b, r, sl] + pe_v[b, r, sl]

        lax.fori_loop(0, _C, lambda r, _: (add_row(r), 0)[1], 0,
                      unroll=2)

        wb[j] = pltpu.async_copy(rows_v.at[b],
                                 out_hbm.at[pl.ds(base0 + j * _C, _C)],
                                 sem_w.at[b])
    wb[_NCHUNK - 1].wait()


_sc_call = pl.kernel(
    _body,
    out_type=jax.ShapeDtypeStruct((_SEQ, _D), jnp.float32),
    mesh=plsc.VectorSubcoreMesh(core_axis_name="c", subcore_axis_name="s"),
    scratch_types=[
        pltpu.VMEM((_BPW,), jnp.int32),
        pltpu.VMEM((2, _C, _D), jnp.float32),
        pltpu.VMEM((2, _C, _D), jnp.float32),
        pltpu.SemaphoreType.DMA((2,)),
        pltpu.SemaphoreType.DMA((2,)),
        pltpu.SemaphoreType.DMA((2,)),
    ],
)


_jit_call = jax.jit(
    lambda x, table, pe: _sc_call(x.astype(jnp.int32), table, pe))

_pe_dev = None


def kernel(x, table):
    global _pe_dev
    if _pe_dev is None:
        _pe_dev = jnp.asarray(_PE)
    return _jit_call(x, table, _pe_dev)
